# baseline (speedup 1.0000x reference)
import jax
import jax.numpy as jnp
from jax import lax
from jax.experimental import pallas as pl
from jax.experimental.pallas import tpu as pltpu

B, F, D = 64, 2048, 128
D3 = 3 * D
CH = 256
NC = F // CH


def _body(nb_ref, q_ref, fr_ref, fa1_ref, fa2_ref, out_ref, acc_ref):
    b = pl.program_id(0)
    c = pl.program_id(1)
    n = nb_ref[b]
    lastc = (n - 1) // CH

    @pl.when(c <= lastc)
    def _():
        q = q_ref[0]
        f = jnp.concatenate(
            [fr_ref[0], fa1_ref[0], fa2_ref[0]], axis=1)
        p = f * (f - 2.0 * q)
        ones = jnp.ones((1, D3), jnp.float32)
        s = lax.dot_general(ones, p, (((1,), (1,)), ((), ())),
                            preferred_element_type=jnp.float32)

        @pl.when(c < lastc)
        def _():
            @pl.when(c == 0)
            def _():
                acc_ref[...] = s

            @pl.when(c > 0)
            def _():
                acc_ref[...] = jnp.minimum(acc_ref[...], s)

        @pl.when(c == lastc)
        def _():
            gidx = c * CH + lax.broadcasted_iota(jnp.int32, (1, CH), 1)
            sm = jnp.where(gidx < n, s, jnp.inf)
            tot = jnp.where(c > 0, jnp.minimum(acc_ref[...], sm), sm)
            nq = jnp.sum(q * q)
            mn = jnp.min(tot, axis=1, keepdims=True) + nq
            out_ref[0] = jnp.exp(-0.5 * mn)


def kernel(rel, arg1, arg2, facts_rel, facts_arg1, facts_arg2, nb_facts):
    def fact_map(b, c, nb):
        return (b, jnp.minimum(c, (nb[b] - 1) // CH), 0)

    grid_spec = pltpu.PrefetchScalarGridSpec(
        num_scalar_prefetch=1,
        grid=(B, NC),
        in_specs=[
            pl.BlockSpec((1, 1, D3), lambda b, c, nb: (b, 0, 0)),
            pl.BlockSpec((1, CH, D), fact_map),
            pl.BlockSpec((1, CH, D), fact_map),
            pl.BlockSpec((1, CH, D), fact_map),
        ],
        out_specs=pl.BlockSpec((1, 1, 1), lambda b, c, nb: (b, 0, 0)),
        scratch_shapes=[pltpu.VMEM((1, CH), jnp.float32)],
    )
    qcat = jnp.concatenate([rel, arg1, arg2], axis=1).reshape(B, 1, D3)
    out = pl.pallas_call(
        _body,
        grid_spec=grid_spec,
        out_shape=jax.ShapeDtypeStruct((B, 1, 1), jnp.float32),
    )(nb_facts, qcat, facts_rel, facts_arg1, facts_arg2)
    return out.reshape(B)


# speedup vs baseline: 1.9456x; 1.9456x over previous
import jax
import jax.numpy as jnp
from jax import lax
from jax.experimental import pallas as pl
from jax.experimental.pallas import tpu as pltpu

B, F, D = 64, 2048, 128
D3 = 3 * D
CH = 512
NSLOT = 3
PREFETCH = 2


def _body(nb_ref, q_ref, fr_hbm, fa1_hbm, fa2_hbm, out_ref,
          bufr, buf1, buf2, sems):
    b = pl.program_id(0)
    n = nb_ref[b]
    trips = (n + CH - 1) // CH
    lastc = trips - 1

    def copies(c, slot):
        src = [fr_hbm, fa1_hbm, fa2_hbm]
        dst = [bufr, buf1, buf2]
        return [
            pltpu.make_async_copy(
                src[i].at[b, pl.ds(c * CH, CH), :], dst[i].at[slot],
                sems.at[i, slot])
            for i in range(3)
        ]

    def start(c, slot):
        for cp in copies(c, slot):
            cp.start()

    def wait(c, slot):
        for cp in copies(c, slot):
            cp.wait()

    for k in range(PREFETCH):
        @pl.when(k < trips)
        def _():
            start(k, k % NSLOT)

    q2 = q_ref[0] * 2.0

    def chunk_sums(slot):
        def part(buf, lo):
            x = buf[slot]
            return x * (x - q2[:, lo:lo + D])

        psum = part(bufr, 0) + part(buf1, D) + part(buf2, 2 * D)
        return jnp.sum(psum, axis=1, keepdims=True)

    def chunk_body(c, acc):
        slot = lax.rem(c, NSLOT)
        pf = c + PREFETCH

        @pl.when(pf < trips)
        def _():
            start(pf, lax.rem(pf, NSLOT))

        wait(c, slot)
        s = chunk_sums(slot)
        return jnp.minimum(acc, jnp.min(s, axis=0, keepdims=True))

    inf11 = jnp.full((1, 1), jnp.inf, jnp.float32)
    acc = lax.fori_loop(0, lastc, chunk_body, inf11)

    slot = lax.rem(lastc, NSLOT)
    wait(lastc, slot)
    s = chunk_sums(slot)
    gidx = lastc * CH + lax.broadcasted_iota(jnp.int32, (CH, 1), 0)
    sm = jnp.where(gidx < n, s, jnp.inf)
    acc = jnp.minimum(acc, jnp.min(sm, axis=0, keepdims=True))

    nq = jnp.sum(q_ref[0] * q_ref[0])
    out_ref[0] = jnp.exp(-0.5 * (acc + nq))


def kernel(rel, arg1, arg2, facts_rel, facts_arg1, facts_arg2, nb_facts):
    grid_spec = pltpu.PrefetchScalarGridSpec(
        num_scalar_prefetch=1,
        grid=(B,),
        in_specs=[
            pl.BlockSpec((1, 1, D3), lambda b, nb: (b, 0, 0)),
            pl.BlockSpec(memory_space=pltpu.MemorySpace.HBM),
            pl.BlockSpec(memory_space=pltpu.MemorySpace.HBM),
            pl.BlockSpec(memory_space=pltpu.MemorySpace.HBM),
        ],
        out_specs=pl.BlockSpec((1, 1, 1), lambda b, nb: (b, 0, 0)),
        scratch_shapes=[
            pltpu.VMEM((NSLOT, CH, D), jnp.float32),
            pltpu.VMEM((NSLOT, CH, D), jnp.float32),
            pltpu.VMEM((NSLOT, CH, D), jnp.float32),
            pltpu.SemaphoreType.DMA((3, NSLOT)),
        ],
    )
    qcat = jnp.concatenate([rel, arg1, arg2], axis=1).reshape(B, 1, D3)
    out = pl.pallas_call(
        _body,
        grid_spec=grid_spec,
        out_shape=jax.ShapeDtypeStruct((B, 1, 1), jnp.float32),
    )(nb_facts, qcat, facts_rel, facts_arg1, facts_arg2)
    return out.reshape(B)
